# baseline (speedup 1.0000x reference)
import jax
import jax.numpy as jnp
from jax import lax
from jax.experimental import pallas as pl
from jax.experimental.pallas import tpu as pltpu


_EPS2 = 1e-24

_TN_CAP = 4096


def _adaptive_avg_pool2d(x, out_hw):
    b, c, h, w = x.shape
    oh, ow = out_hw
    if h == oh and w == ow:
        return x
    assert h % oh == 0 and w % ow == 0, "adaptive pool requires divisible sizes"
    return x.reshape(b, c, oh, h // oh, ow, w // ow).mean(axis=(3, 5))


def _normalized_gram_sse(gs, gt):
    eps2 = jnp.float32(_EPS2)
    rows = lax.broadcasted_iota(jnp.int32, gs.shape, 1)
    cols = lax.broadcasted_iota(jnp.int32, gs.shape, 2)
    eye = rows == cols
    gs_d = jnp.where(eye, gs, 0.0)
    gt_d = jnp.where(eye, gt, 0.0)
    inv_s_r = lax.rsqrt(jnp.maximum(jnp.sum(gs_d, axis=2, keepdims=True), eps2))
    inv_s_c = lax.rsqrt(jnp.maximum(jnp.sum(gs_d, axis=1, keepdims=True), eps2))
    inv_t_r = lax.rsqrt(jnp.maximum(jnp.sum(gt_d, axis=2, keepdims=True), eps2))
    inv_t_c = lax.rsqrt(jnp.maximum(jnp.sum(gt_d, axis=1, keepdims=True), eps2))
    diff = (gs * inv_s_r) * inv_s_c - (gt * inv_t_r) * inv_t_c
    return jnp.sum(diff * diff)


_BDN = (((2,), (2,)), ((0,), (0,)))


def _make_loss_kernel(inv_denom):
    def body(fs_ref, ft_ref, out_ref):
        i = pl.program_id(0)

        @pl.when(i == 0)
        def _init():
            out_ref[0, 0] = jnp.float32(0.0)

        fs = fs_ref[...].astype(jnp.bfloat16)
        ft = ft_ref[...].astype(jnp.bfloat16)
        gs = lax.dot_general(fs, fs, dimension_numbers=_BDN,
                             preferred_element_type=jnp.float32)
        gt = lax.dot_general(ft, ft, dimension_numbers=_BDN,
                             preferred_element_type=jnp.float32)
        out_ref[0, 0] += _normalized_gram_sse(gs, gt)

        @pl.when(i == pl.num_programs(0) - 1)
        def _finish():
            out_ref[0, 0] = out_ref[0, 0] * jnp.float32(inv_denom)

    return body


def _make_tiled_kernel(n_total, tn, inv_denom):
    needs_mask = (n_total % tn) != 0

    def body(fs_ref, ft_ref, out_ref, gs_acc, gt_acc):
        n = pl.program_id(1)

        @pl.when((pl.program_id(0) == 0) & (n == 0))
        def _init():
            out_ref[0, 0] = jnp.float32(0.0)

        @pl.when(n == 0)
        def _zero():
            gs_acc[...] = jnp.zeros_like(gs_acc)
            gt_acc[...] = jnp.zeros_like(gt_acc)

        fs = fs_ref[...]
        ft = ft_ref[...]
        if needs_mask:
            rem = jnp.int32(n_total) - n * jnp.int32(tn)
            keep = lax.broadcasted_iota(jnp.int32, fs.shape, 2) < rem
            fs = jnp.where(keep, fs, jnp.zeros_like(fs))
            ft = jnp.where(keep, ft, jnp.zeros_like(ft))
        fs = fs.astype(jnp.bfloat16)
        ft = ft.astype(jnp.bfloat16)
        gs_acc[...] += lax.dot_general(fs, fs, dimension_numbers=_BDN,
                                       preferred_element_type=jnp.float32)
        gt_acc[...] += lax.dot_general(ft, ft, dimension_numbers=_BDN,
                                       preferred_element_type=jnp.float32)

        @pl.when(n == pl.num_programs(1) - 1)
        def _fold():
            out_ref[0, 0] += _normalized_gram_sse(gs_acc[...], gt_acc[...])

        @pl.when((pl.program_id(0) == pl.num_programs(0) - 1)
                 & (n == pl.num_programs(1) - 1))
        def _finish():
            out_ref[0, 0] = out_ref[0, 0] * jnp.float32(inv_denom)

    return body


def kernel(g_s, g_t):
    if g_s.shape[2] > g_t.shape[2]:
        g_s = _adaptive_avg_pool2d(g_s, g_t.shape[2:])
    elif g_s.shape[2] < g_t.shape[2]:
        g_t = _adaptive_avg_pool2d(g_t, g_s.shape[2:])

    b, c, h, w = g_s.shape
    n = h * w
    fs = g_s.reshape(b, c, n)
    ft = g_t.reshape(b, c, n)
    inv_denom = 1.0 / float(b * c * c)

    bb = next(x for x in (8, 4, 2, 1) if b % x == 0)

    if n <= _TN_CAP:
        out = pl.pallas_call(
            _make_loss_kernel(inv_denom),
            out_shape=jax.ShapeDtypeStruct((1, 1), jnp.float32),
            grid=(b // bb,),
            in_specs=[
                pl.BlockSpec((bb, c, n), lambda i: (i, 0, 0)),
                pl.BlockSpec((bb, c, n), lambda i: (i, 0, 0)),
            ],
            out_specs=pl.BlockSpec(memory_space=pltpu.SMEM),
            compiler_params=pltpu.CompilerParams(
                dimension_semantics=("arbitrary",)),
        )(fs, ft)
    else:
        tn = _TN_CAP
        out = pl.pallas_call(
            _make_tiled_kernel(n, tn, inv_denom),
            out_shape=jax.ShapeDtypeStruct((1, 1), jnp.float32),
            grid=(b // bb, pl.cdiv(n, tn)),
            in_specs=[
                pl.BlockSpec((bb, c, tn), lambda i, j: (i, 0, j)),
                pl.BlockSpec((bb, c, tn), lambda i, j: (i, 0, j)),
            ],
            out_specs=pl.BlockSpec(memory_space=pltpu.SMEM),
            scratch_shapes=[pltpu.VMEM((bb, c, c), jnp.float32),
                            pltpu.VMEM((bb, c, c), jnp.float32)],
            compiler_params=pltpu.CompilerParams(
                dimension_semantics=("arbitrary", "arbitrary")),
        )(fs, ft)

    return out.reshape(())


# speedup vs baseline: 1.0094x; 1.0094x over previous
import jax
import jax.numpy as jnp
from jax import lax
from jax.experimental import pallas as pl
from jax.experimental.pallas import tpu as pltpu


_EPS2 = 1e-24

_TN_CAP = 4096


def _adaptive_avg_pool2d(x, out_hw):
    b, c, h, w = x.shape
    oh, ow = out_hw
    if h == oh and w == ow:
        return x
    assert h % oh == 0 and w % ow == 0, "adaptive pool requires divisible sizes"
    return x.reshape(b, c, oh, h // oh, ow, w // ow).mean(axis=(3, 5))


def _normalized_gram_sse(gs, gt):
    eps2 = jnp.float32(_EPS2)
    rows = lax.broadcasted_iota(jnp.int32, gs.shape, 1)
    cols = lax.broadcasted_iota(jnp.int32, gs.shape, 2)
    eye = rows == cols
    gs_d = jnp.where(eye, gs, 0.0)
    gt_d = jnp.where(eye, gt, 0.0)
    inv_s_r = lax.rsqrt(jnp.maximum(jnp.sum(gs_d, axis=2, keepdims=True), eps2))
    inv_s_c = lax.rsqrt(jnp.maximum(jnp.sum(gs_d, axis=1, keepdims=True), eps2))
    inv_t_r = lax.rsqrt(jnp.maximum(jnp.sum(gt_d, axis=2, keepdims=True), eps2))
    inv_t_c = lax.rsqrt(jnp.maximum(jnp.sum(gt_d, axis=1, keepdims=True), eps2))
    diff = (gs * inv_s_r) * inv_s_c - (gt * inv_t_r) * inv_t_c
    return jnp.sum(diff * diff)


_BDN = (((2,), (2,)), ((0,), (0,)))


def _make_loss_kernel(inv_denom):
    def body(fs_ref, ft_ref, out_ref):
        i = pl.program_id(0)

        @pl.when(i == 0)
        def _init():
            out_ref[0, 0] = jnp.float32(0.0)

        fs = fs_ref[...].astype(jnp.bfloat16)
        ft = ft_ref[...].astype(jnp.bfloat16)
        gs = lax.dot_general(fs, fs, dimension_numbers=_BDN,
                             preferred_element_type=jnp.float32)
        gt = lax.dot_general(ft, ft, dimension_numbers=_BDN,
                             preferred_element_type=jnp.float32)
        out_ref[0, 0] += _normalized_gram_sse(gs, gt)

        @pl.when(i == pl.num_programs(0) - 1)
        def _finish():
            out_ref[0, 0] = out_ref[0, 0] * jnp.float32(inv_denom)

    return body


def _make_tiled_kernel(n_total, tn, inv_denom):
    needs_mask = (n_total % tn) != 0

    def body(fs_ref, ft_ref, out_ref, gs_acc, gt_acc):
        n = pl.program_id(1)

        @pl.when((pl.program_id(0) == 0) & (n == 0))
        def _init():
            out_ref[0, 0] = jnp.float32(0.0)

        @pl.when(n == 0)
        def _zero():
            gs_acc[...] = jnp.zeros_like(gs_acc)
            gt_acc[...] = jnp.zeros_like(gt_acc)

        fs = fs_ref[...]
        ft = ft_ref[...]
        if needs_mask:
            rem = jnp.int32(n_total) - n * jnp.int32(tn)
            keep = lax.broadcasted_iota(jnp.int32, fs.shape, 2) < rem
            fs = jnp.where(keep, fs, jnp.zeros_like(fs))
            ft = jnp.where(keep, ft, jnp.zeros_like(ft))
        fs = fs.astype(jnp.bfloat16)
        ft = ft.astype(jnp.bfloat16)
        gs_acc[...] += lax.dot_general(fs, fs, dimension_numbers=_BDN,
                                       preferred_element_type=jnp.float32)
        gt_acc[...] += lax.dot_general(ft, ft, dimension_numbers=_BDN,
                                       preferred_element_type=jnp.float32)

        @pl.when(n == pl.num_programs(1) - 1)
        def _fold():
            out_ref[0, 0] += _normalized_gram_sse(gs_acc[...], gt_acc[...])

        @pl.when((pl.program_id(0) == pl.num_programs(0) - 1)
                 & (n == pl.num_programs(1) - 1))
        def _finish():
            out_ref[0, 0] = out_ref[0, 0] * jnp.float32(inv_denom)

    return body


def kernel(g_s, g_t):
    if g_s.shape[2] > g_t.shape[2]:
        g_s = _adaptive_avg_pool2d(g_s, g_t.shape[2:])
    elif g_s.shape[2] < g_t.shape[2]:
        g_t = _adaptive_avg_pool2d(g_t, g_s.shape[2:])

    b, c, h, w = g_s.shape
    n = h * w
    fs = g_s.reshape(b, c, n)
    ft = g_t.reshape(b, c, n)
    inv_denom = 1.0 / float(b * c * c)

    bb = next(x for x in (4, 2, 1) if b % x == 0)

    if n <= _TN_CAP:
        out = pl.pallas_call(
            _make_loss_kernel(inv_denom),
            out_shape=jax.ShapeDtypeStruct((1, 1), jnp.float32),
            grid=(b // bb,),
            in_specs=[
                pl.BlockSpec((bb, c, n), lambda i: (i, 0, 0)),
                pl.BlockSpec((bb, c, n), lambda i: (i, 0, 0)),
            ],
            out_specs=pl.BlockSpec(memory_space=pltpu.SMEM),
            compiler_params=pltpu.CompilerParams(
                dimension_semantics=("arbitrary",)),
        )(fs, ft)
    else:
        tn = _TN_CAP
        out = pl.pallas_call(
            _make_tiled_kernel(n, tn, inv_denom),
            out_shape=jax.ShapeDtypeStruct((1, 1), jnp.float32),
            grid=(b // bb, pl.cdiv(n, tn)),
            in_specs=[
                pl.BlockSpec((bb, c, tn), lambda i, j: (i, 0, j)),
                pl.BlockSpec((bb, c, tn), lambda i, j: (i, 0, j)),
            ],
            out_specs=pl.BlockSpec(memory_space=pltpu.SMEM),
            scratch_shapes=[pltpu.VMEM((bb, c, c), jnp.float32),
                            pltpu.VMEM((bb, c, c), jnp.float32)],
            compiler_params=pltpu.CompilerParams(
                dimension_semantics=("arbitrary", "arbitrary")),
        )(fs, ft)

    return out.reshape(())


# speedup vs baseline: 2.9162x; 2.8889x over previous
import jax
import jax.numpy as jnp
from jax import lax
from jax.experimental import pallas as pl
from jax.experimental.pallas import tpu as pltpu


_EPS2 = 1e-24

_TN_CAP = 4096


def _adaptive_avg_pool2d(x, out_hw):
    b, c, h, w = x.shape
    oh, ow = out_hw
    if h == oh and w == ow:
        return x
    assert h % oh == 0 and w % ow == 0, "adaptive pool requires divisible sizes"
    return x.reshape(b, c, oh, h // oh, ow, w // ow).mean(axis=(3, 5))


def _normalized_gram_sse(gs, gt):
    eps2 = jnp.float32(_EPS2)
    rows = lax.broadcasted_iota(jnp.int32, gs.shape, 1)
    cols = lax.broadcasted_iota(jnp.int32, gs.shape, 2)
    eye = rows == cols
    gs_d = jnp.where(eye, gs, 0.0)
    gt_d = jnp.where(eye, gt, 0.0)
    inv_s_r = lax.rsqrt(jnp.maximum(jnp.sum(gs_d, axis=2, keepdims=True), eps2))
    inv_s_c = lax.rsqrt(jnp.maximum(jnp.sum(gs_d, axis=1, keepdims=True), eps2))
    inv_t_r = lax.rsqrt(jnp.maximum(jnp.sum(gt_d, axis=2, keepdims=True), eps2))
    inv_t_c = lax.rsqrt(jnp.maximum(jnp.sum(gt_d, axis=1, keepdims=True), eps2))
    diff = (gs * inv_s_r) * inv_s_c - (gt * inv_t_r) * inv_t_c
    return jnp.sum(diff * diff)


_BDN = (((2,), (2,)), ((0,), (0,)))
_TDN = (((0,), (0,)), ((), ()))


def _normalized_gram_sse2(gs, gt):
    eps2 = jnp.float32(_EPS2)
    rows = lax.broadcasted_iota(jnp.int32, gs.shape, 0)
    cols = lax.broadcasted_iota(jnp.int32, gs.shape, 1)
    eye = rows == cols
    gs_d = jnp.where(eye, gs, 0.0)
    gt_d = jnp.where(eye, gt, 0.0)
    inv_s_r = lax.rsqrt(jnp.maximum(jnp.sum(gs_d, axis=1, keepdims=True), eps2))
    inv_s_c = lax.rsqrt(jnp.maximum(jnp.sum(gs_d, axis=0, keepdims=True), eps2))
    inv_t_r = lax.rsqrt(jnp.maximum(jnp.sum(gt_d, axis=1, keepdims=True), eps2))
    inv_t_c = lax.rsqrt(jnp.maximum(jnp.sum(gt_d, axis=0, keepdims=True), eps2))
    diff = (gs * inv_s_r) * inv_s_c - (gt * inv_t_r) * inv_t_c
    return jnp.sum(diff * diff)


def _make_nbc_loss_kernel(inv_denom, bb):
    def body(fs_ref, ft_ref, out_ref):
        i = pl.program_id(0)

        @pl.when(i == 0)
        def _init():
            out_ref[0, 0] = jnp.float32(0.0)

        fs = jnp.transpose(fs_ref[...].astype(jnp.bfloat16), (1, 0, 2))
        ft = jnp.transpose(ft_ref[...].astype(jnp.bfloat16), (1, 0, 2))
        sse = jnp.float32(0.0)
        for j in range(bb):
            gs = lax.dot_general(fs[j], fs[j], dimension_numbers=_TDN,
                                 preferred_element_type=jnp.float32)
            gt = lax.dot_general(ft[j], ft[j], dimension_numbers=_TDN,
                                 preferred_element_type=jnp.float32)
            sse += _normalized_gram_sse2(gs, gt)
        out_ref[0, 0] += sse

        @pl.when(i == pl.num_programs(0) - 1)
        def _finish():
            out_ref[0, 0] = out_ref[0, 0] * jnp.float32(inv_denom)

    return body


def _make_tiled_kernel(n_total, tn, inv_denom):
    needs_mask = (n_total % tn) != 0

    def body(fs_ref, ft_ref, out_ref, gs_acc, gt_acc):
        n = pl.program_id(1)

        @pl.when((pl.program_id(0) == 0) & (n == 0))
        def _init():
            out_ref[0, 0] = jnp.float32(0.0)

        @pl.when(n == 0)
        def _zero():
            gs_acc[...] = jnp.zeros_like(gs_acc)
            gt_acc[...] = jnp.zeros_like(gt_acc)

        fs = fs_ref[...]
        ft = ft_ref[...]
        if needs_mask:
            rem = jnp.int32(n_total) - n * jnp.int32(tn)
            keep = lax.broadcasted_iota(jnp.int32, fs.shape, 2) < rem
            fs = jnp.where(keep, fs, jnp.zeros_like(fs))
            ft = jnp.where(keep, ft, jnp.zeros_like(ft))
        fs = fs.astype(jnp.bfloat16)
        ft = ft.astype(jnp.bfloat16)
        gs_acc[...] += lax.dot_general(fs, fs, dimension_numbers=_BDN,
                                       preferred_element_type=jnp.float32)
        gt_acc[...] += lax.dot_general(ft, ft, dimension_numbers=_BDN,
                                       preferred_element_type=jnp.float32)

        @pl.when(n == pl.num_programs(1) - 1)
        def _fold():
            out_ref[0, 0] += _normalized_gram_sse(gs_acc[...], gt_acc[...])

        @pl.when((pl.program_id(0) == pl.num_programs(0) - 1)
                 & (n == pl.num_programs(1) - 1))
        def _finish():
            out_ref[0, 0] = out_ref[0, 0] * jnp.float32(inv_denom)

    return body


def kernel(g_s, g_t):
    if g_s.shape[2] > g_t.shape[2]:
        g_s = _adaptive_avg_pool2d(g_s, g_t.shape[2:])
    elif g_s.shape[2] < g_t.shape[2]:
        g_t = _adaptive_avg_pool2d(g_t, g_s.shape[2:])

    b, c, h, w = g_s.shape
    n = h * w
    inv_denom = 1.0 / float(b * c * c)

    nbc_ok = (b % 8 == 0 and c % 128 == 0
              and 2 * 2 * 4 * n * 8 * c <= (44 << 20))
    if nbc_ok:
        bb = 8
        fs = g_s.transpose(2, 3, 0, 1).reshape(n, b, c)
        ft = g_t.transpose(2, 3, 0, 1).reshape(n, b, c)
        out = pl.pallas_call(
            _make_nbc_loss_kernel(inv_denom, bb),
            out_shape=jax.ShapeDtypeStruct((1, 1), jnp.float32),
            grid=(b // bb,),
            in_specs=[
                pl.BlockSpec((n, bb, c), lambda i: (0, i, 0)),
                pl.BlockSpec((n, bb, c), lambda i: (0, i, 0)),
            ],
            out_specs=pl.BlockSpec(memory_space=pltpu.SMEM),
            compiler_params=pltpu.CompilerParams(
                dimension_semantics=("arbitrary",)),
        )(fs, ft)
    else:
        tn = min(_TN_CAP, n)
        fs = g_s.reshape(b, c, n)
        ft = g_t.reshape(b, c, n)
        bb = next(x for x in (4, 2, 1) if b % x == 0)
        out = pl.pallas_call(
            _make_tiled_kernel(n, tn, inv_denom),
            out_shape=jax.ShapeDtypeStruct((1, 1), jnp.float32),
            grid=(b // bb, pl.cdiv(n, tn)),
            in_specs=[
                pl.BlockSpec((bb, c, tn), lambda i, j: (i, 0, j)),
                pl.BlockSpec((bb, c, tn), lambda i, j: (i, 0, j)),
            ],
            out_specs=pl.BlockSpec(memory_space=pltpu.SMEM),
            scratch_shapes=[pltpu.VMEM((bb, c, c), jnp.float32),
                            pltpu.VMEM((bb, c, c), jnp.float32)],
            compiler_params=pltpu.CompilerParams(
                dimension_semantics=("arbitrary", "arbitrary")),
        )(fs, ft)

    return out.reshape(())
